# Initial kernel scaffold; baseline (speedup 1.0000x reference)
import jax
import jax.numpy as jnp
from jax.experimental import pallas as pl


def kernel(idxes, table):
    raise NotImplementedError("write your pallas kernel here")


# speedup vs baseline: 1.0302x; 1.0302x over previous
import functools

import jax
import jax.numpy as jnp
from jax import lax
from jax.experimental import pallas as pl
from jax.experimental.pallas import tpu as pltpu
from jax.experimental.pallas import tpu_sc as plsc

_DIM = 300
_DIMP = 384
_NC = 2
_NS = 16
_NW = _NC * _NS
_CH = 128


def _make_gather(n_idx):
    assert n_idx % (_NW * _CH) == 0
    n_chunk = n_idx // (_NW * _CH)
    b_per_w = n_chunk * _CH
    mesh = plsc.VectorSubcoreMesh(core_axis_name="c", subcore_axis_name="s")

    @functools.partial(
        pl.kernel,
        mesh=mesh,
        out_type=jax.ShapeDtypeStruct((n_idx, _DIMP), jnp.float32),
        scratch_types=[
            pltpu.VMEM((n_chunk, _CH), jnp.int32),
            pltpu.VMEM((_CH, _DIMP), jnp.float32),
            pltpu.SemaphoreType.DMA,
        ],
    )
    def gather_kernel(idx_hbm, table_hbm, out_hbm, idx_v, rows_v, gsem):
        wid = lax.axis_index("s") * _NC + lax.axis_index("c")
        base = pl.multiple_of(wid * b_per_w, 8)
        pltpu.sync_copy(idx_hbm.at[wid], idx_v)

        def body(j, carry):
            pltpu.async_copy(table_hbm.at[idx_v.at[j]], rows_v, gsem).wait()
            row0 = pl.multiple_of(base + j * _CH, 8)
            pltpu.sync_copy(rows_v, out_hbm.at[pl.ds(row0, _CH)])
            return carry

        lax.fori_loop(0, n_chunk, body, 0)

    return gather_kernel


def kernel(idxes, table):
    batch, seq = idxes.shape
    n_idx = batch * seq
    idx3d = idxes.reshape(_NW, n_idx // (_NW * _CH), _CH).astype(jnp.int32)
    table_p = jnp.pad(table, ((0, 0), (0, _DIMP - _DIM)))
    out = _make_gather(n_idx)(idx3d, table_p)
    return out[:, :_DIM].reshape(batch, seq, _DIM)
